# baseline (speedup 1.0000x reference)
import functools

import jax
import jax.numpy as jnp
from jax import lax
from jax.experimental import pallas as pl
from jax.experimental.pallas import tpu as pltpu
from jax.experimental.pallas import tpu_sc as plsc

N = 10000
E = 160000
D_IN = 128
D_EDGE = 16
D = 32
EH = 32
STEPS = 6
S2S_STEPS = 6

NC, NS = 2, 16
NW = NC * NS
E_PAD = 163840
EPW = E_PAD // NW
G_OUT = 5
IDX_R = 8
ROWS_CH = IDX_R * 128
HALF = ROWS_CH // 2
N_ACC = 10112
ZR = N_ACC // NS

_sc_mesh = plsc.VectorSubcoreMesh(core_axis_name="c", subcore_axis_name="s")


@functools.partial(
    pl.kernel,
    mesh=_sc_mesh,
    out_type=jax.ShapeDtypeStruct((E_PAD, 128), jnp.float32),
    scratch_types=[
        pltpu.VMEM((IDX_R, 128), jnp.int32),
        pltpu.VMEM((HALF, 128), jnp.float32),
        pltpu.SemaphoreType.DMA,
    ],
)
def _sc_gather(table_hbm, idx_hbm, out_hbm, idx_v, rows_v, sem):
    c = lax.axis_index("c")
    s = lax.axis_index("s")
    wid = s * NC + c
    base_row = wid * (EPW // 128)

    def outer(g, _):
        irow = base_row + g * IDX_R
        pltpu.sync_copy(idx_hbm.at[pl.ds(irow, IDX_R)], idx_v)

        def half_loop(hf, _):
            def inner(r4, _):
                pltpu.async_copy(table_hbm.at[idx_v.at[hf * 4 + r4]],
                                 rows_v.at[pl.ds(r4 * 128, 128)], sem).wait()
                return 0
            lax.fori_loop(0, 4, inner, 0)
            pltpu.sync_copy(
                rows_v, out_hbm.at[pl.ds(irow * 128 + hf * HALF, HALF)])
            return 0

        lax.fori_loop(0, 2, half_loop, 0)
        return 0

    lax.fori_loop(0, G_OUT, outer, 0)


N_AGG = 10240
RW = N_AGG // NW
SCH = 512


T_SEG = 2048
W_SEG = 4


def _seg_body(sb_ref, ids_ref, msg_ref, o_ref):
    w = pl.program_id(0)
    j = pl.program_id(1)
    base = w * RW
    idv = ids_ref[0]
    row_iota = lax.broadcasted_iota(jnp.int32, (RW, T_SEG), 0)
    sel = (idv == base + row_iota).astype(jnp.bfloat16)
    part = jnp.dot(sel, msg_ref[...].astype(jnp.bfloat16),
                   preferred_element_type=jnp.float32)

    @pl.when(j == 0)
    def _():
        o_ref[...] = part

    @pl.when(j > 0)
    def _():
        o_ref[...] = o_ref[...] + part


def _seg(msg, ids3, sb):
    grid_spec = pltpu.PrefetchScalarGridSpec(
        num_scalar_prefetch=1,
        grid=(NW, W_SEG),
        in_specs=[
            pl.BlockSpec((1, 1, T_SEG), lambda w, j, sb: (sb[w] + j, 0, 0)),
            pl.BlockSpec((T_SEG, D), lambda w, j, sb: (sb[w] + j, 0)),
        ],
        out_specs=pl.BlockSpec((RW, D), lambda w, j, sb: (w, 0)),
    )
    return pl.pallas_call(
        _seg_body,
        grid_spec=grid_spec,
        out_shape=jax.ShapeDtypeStruct((N_AGG, D), jnp.float32),
    )(sb, ids3, msg)


def _lin0_body(x_ref, w_ref, b_ref, o_ref):
    out = jax.nn.relu(
        jnp.dot(x_ref[...], w_ref[...], preferred_element_type=jnp.float32)
        + b_ref[...][0:1, :])
    o_ref[...] = jnp.tile(out, (1, 4))


def _lin0(x, wT, b8):
    return pl.pallas_call(
        _lin0_body,
        out_shape=jax.ShapeDtypeStruct((N, 128), jnp.float32),
    )(x, wT, b8)


def _eh_body(a_ref, w_ref, b_ref, o_ref):
    o_ref[...] = jax.nn.relu(
        jnp.dot(a_ref[...], w_ref[...], preferred_element_type=jnp.float32)
        + b_ref[...][0:1, :])


def _ehk(ea_p, w1T, b8):
    blk = E_PAD // 8
    return pl.pallas_call(
        _eh_body,
        grid=(8,),
        in_specs=[
            pl.BlockSpec((blk, D_EDGE), lambda i: (i, 0)),
            pl.BlockSpec((D_EDGE, EH), lambda i: (0, 0)),
            pl.BlockSpec((8, EH), lambda i: (0, 0)),
        ],
        out_specs=pl.BlockSpec((blk, EH), lambda i: (i, 0)),
        out_shape=jax.ShapeDtypeStruct((E_PAD, EH), jnp.float32),
    )(ea_p, w1T, b8)


T_MSG = 2048


def _msg_body(eh_ref, xs_ref, bm_ref, b2r_ref, o_ref):
    ehb = eh_ref[...]
    xsw = xs_ref[...]
    z = jnp.repeat(ehb, D, axis=1) * jnp.tile(xsw, (1, EH // 4))
    o_ref[...] = (
        jnp.dot(z, bm_ref[...], preferred_element_type=jnp.float32)
        + jnp.dot(xsw[:, 0:D], b2r_ref[...],
                  preferred_element_type=jnp.float32))


def _msg(eh_p, xs, bmat, b2r):
    return pl.pallas_call(
        _msg_body,
        grid=(E_PAD // T_MSG,),
        in_specs=[
            pl.BlockSpec((T_MSG, EH), lambda i: (i, 0)),
            pl.BlockSpec((T_MSG, 128), lambda i: (i, 0)),
            pl.BlockSpec((EH * D, D), lambda i: (0, 0)),
            pl.BlockSpec((D, D), lambda i: (0, 0)),
        ],
        out_specs=pl.BlockSpec((T_MSG, D), lambda i: (i, 0)),
        out_shape=jax.ShapeDtypeStruct((E_PAD, D), jnp.float32),
    )(eh_p, xs, bmat, b2r)


T_GRU = 2000


def _gru_body(agg_ref, hw_ref, wih_ref, whh_ref, bih_ref, bhh_ref,
              cb_ref, o_ref):
    m = jax.nn.relu(agg_ref[...] + cb_ref[...][0:1, :])
    gi = jnp.dot(m, wih_ref[...],
                 preferred_element_type=jnp.float32) + bih_ref[...][0:1, :]
    h = hw_ref[...][:, 0:D]
    gh = jnp.dot(h, whh_ref[...],
                 preferred_element_type=jnp.float32) + bhh_ref[...][0:1, :]
    r = jax.nn.sigmoid(gi[:, 0:D] + gh[:, 0:D])
    zg = jax.nn.sigmoid(gi[:, D:2 * D] + gh[:, D:2 * D])
    n = jnp.tanh(gi[:, 2 * D:] + r * gh[:, 2 * D:])
    o_ref[...] = jnp.tile((1.0 - zg) * n + zg * h, (1, 4))


def _gru(agg, hw, wihT, whhT, bih8, bhh8, cb8):
    return pl.pallas_call(
        _gru_body,
        grid=(N // T_GRU,),
        in_specs=[
            pl.BlockSpec((T_GRU, D), lambda i: (i, 0)),
            pl.BlockSpec((T_GRU, 128), lambda i: (i, 0)),
            pl.BlockSpec((D, 3 * D), lambda i: (0, 0)),
            pl.BlockSpec((D, 3 * D), lambda i: (0, 0)),
            pl.BlockSpec((8, 3 * D), lambda i: (0, 0)),
            pl.BlockSpec((8, 3 * D), lambda i: (0, 0)),
            pl.BlockSpec((8, D), lambda i: (0, 0)),
        ],
        out_specs=pl.BlockSpec((T_GRU, 128), lambda i: (i, 0)),
        out_shape=jax.ShapeDtypeStruct((N, 128), jnp.float32),
    )(agg, hw, wihT, whhT, bih8, bhh8, cb8)


def _s2s_body(out_ref, wih0, whh0, b0, wih1, whh1, b1, wih2, whh2, b2,
              w1_ref, rb1_ref, w2_ref, rb2_ref, o_ref):
    outv = out_ref[...][:, 0:D]

    def cell(inp, h, c, wih, whh, b):
        g = (jnp.dot(inp, wih[...], preferred_element_type=jnp.float32)
             + jnp.dot(h, whh[...], preferred_element_type=jnp.float32)
             + b[...][0:1, :])
        gi = g[:, 0:D]
        gf = g[:, D:2 * D]
        gg = g[:, 2 * D:3 * D]
        go = g[:, 3 * D:4 * D]
        c2 = jax.nn.sigmoid(gf) * c + jax.nn.sigmoid(gi) * jnp.tanh(gg)
        h2 = jax.nn.sigmoid(go) * jnp.tanh(c2)
        return h2, c2

    def step(t, carry):
        q_star, h0, c0, h1, c1, h2, c2 = carry
        h0, c0 = cell(q_star, h0, c0, wih0, whh0, b0)
        h1, c1 = cell(h0, h1, c1, wih1, whh1, b1)
        h2, c2 = cell(h1, h2, c2, wih2, whh2, b2)
        q = h2
        e = jnp.sum(outv * q, axis=1, keepdims=True)
        mx = jnp.max(e, axis=0, keepdims=True)
        al = jnp.exp(e - mx)
        ssum = jnp.sum(al, axis=0, keepdims=True)
        readout = jnp.sum(al * outv, axis=0, keepdims=True) / ssum
        q_star = jnp.concatenate([q, readout], axis=1)
        return (q_star, h0, c0, h1, c1, h2, c2)

    z1 = jnp.zeros((1, D), jnp.float32)
    init = (jnp.zeros((1, 2 * D), jnp.float32), z1, z1, z1, z1, z1, z1)
    q_star = lax.fori_loop(0, S2S_STEPS, step, init)[0]
    hmid = jax.nn.relu(
        jnp.dot(q_star, w1_ref[...], preferred_element_type=jnp.float32)
        + rb1_ref[...][0:1, :])
    res = (jnp.dot(hmid, w2_ref[...], preferred_element_type=jnp.float32)
           + rb2_ref[...][0:1, :])
    o_ref[...] = jnp.broadcast_to(res, (8, D))


def _s2s(out, args):
    return pl.pallas_call(
        _s2s_body,
        out_shape=jax.ShapeDtypeStruct((8, D), jnp.float32),
    )(out, *args)


def _pad8(b):
    return jnp.broadcast_to(b[None, :], (8, b.shape[0]))


def kernel(x, edge_index, edge_attr, lin0_W, lin0_b, enet_W1, enet_b1,
           enet_W2, enet_b2, conv_b, gru_Wih, gru_Whh, gru_bih, gru_bhh,
           lstm_Wih_0, lstm_Whh_0, lstm_bih_0, lstm_bhh_0,
           lstm_Wih_1, lstm_Whh_1, lstm_bih_1, lstm_bhh_1,
           lstm_Wih_2, lstm_Whh_2, lstm_bih_2, lstm_bhh_2,
           ro_W1, ro_b1, ro_W2, ro_b2):
    pad = E_PAD - E
    order = jnp.argsort(edge_index[1])
    src_s = edge_index[0][order]
    dst_s = edge_index[1][order]
    ea_s = edge_attr[order]
    src_p = jnp.pad(src_s, (0, pad)).reshape(E_PAD // 128, 128)
    dst_p = jnp.concatenate(
        [dst_s, jnp.full((pad,), N_AGG - 8, jnp.int32)])
    ea_p = jnp.pad(ea_s, ((0, pad), (0, 0)))
    bounds = jnp.searchsorted(
        dst_p, jnp.arange(NW, dtype=jnp.int32) * RW).astype(jnp.int32)
    sb = jnp.minimum(bounds // T_SEG, E_PAD // T_SEG - W_SEG)
    ids3 = dst_p.reshape(E_PAD // T_SEG, 1, T_SEG)

    bmat = enet_W2.reshape(D, D, EH).transpose(2, 0, 1).reshape(EH * D, D)
    b2r = enet_b2.reshape(D, D)

    out = _lin0(x, lin0_W.T, _pad8(lin0_b))
    eh = _ehk(ea_p, enet_W1.T, _pad8(enet_b1))

    gru_args = (gru_Wih.T, gru_Whh.T, _pad8(gru_bih), _pad8(gru_bhh),
                _pad8(conv_b))
    for _ in range(STEPS):
        xs = _sc_gather(out, src_p)
        msg = _msg(eh, xs, bmat, b2r)
        agg = _seg(msg, ids3, sb)
        out = _gru(agg[:N], out, *gru_args)

    s2s_args = (
        lstm_Wih_0.T, lstm_Whh_0.T, _pad8(lstm_bih_0 + lstm_bhh_0),
        lstm_Wih_1.T, lstm_Whh_1.T, _pad8(lstm_bih_1 + lstm_bhh_1),
        lstm_Wih_2.T, lstm_Whh_2.T, _pad8(lstm_bih_2 + lstm_bhh_2),
        ro_W1.T, _pad8(ro_b1), ro_W2.T, _pad8(ro_b2),
    )
    res8 = _s2s(out, s2s_args)
    return res8[0:1]


# speedup vs baseline: 1.0141x; 1.0141x over previous
import functools

import jax
import jax.numpy as jnp
from jax import lax
from jax.experimental import pallas as pl
from jax.experimental.pallas import tpu as pltpu
from jax.experimental.pallas import tpu_sc as plsc

N = 10000
E = 160000
D_IN = 128
D_EDGE = 16
D = 32
EH = 32
STEPS = 6
S2S_STEPS = 6

NC, NS = 2, 16
NW = NC * NS
E_PAD = 163840
EPW = E_PAD // NW
G_OUT = 5
IDX_R = 8
ROWS_CH = IDX_R * 128
HALF = ROWS_CH // 2
N_ACC = 10112
ZR = N_ACC // NS

_sc_mesh = plsc.VectorSubcoreMesh(core_axis_name="c", subcore_axis_name="s")


@functools.partial(
    pl.kernel,
    mesh=_sc_mesh,
    out_type=jax.ShapeDtypeStruct((E_PAD, 128), jnp.float32),
    scratch_types=[
        pltpu.VMEM((IDX_R, 128), jnp.int32),
        pltpu.VMEM((HALF, 128), jnp.float32),
        pltpu.SemaphoreType.DMA,
    ],
)
def _sc_gather(table_hbm, idx_hbm, out_hbm, idx_v, rows_v, sem):
    c = lax.axis_index("c")
    s = lax.axis_index("s")
    wid = s * NC + c
    base_row = wid * (EPW // 128)

    def outer(g, _):
        irow = base_row + g * IDX_R
        pltpu.sync_copy(idx_hbm.at[pl.ds(irow, IDX_R)], idx_v)

        def half_loop(hf, _):
            copies = [
                pltpu.async_copy(table_hbm.at[idx_v.at[hf * 4 + r4]],
                                 rows_v.at[pl.ds(r4 * 128, 128)], sem)
                for r4 in range(4)
            ]
            for cp in copies:
                cp.wait()
            pltpu.sync_copy(
                rows_v, out_hbm.at[pl.ds(irow * 128 + hf * HALF, HALF)])
            return 0

        lax.fori_loop(0, 2, half_loop, 0)
        return 0

    lax.fori_loop(0, G_OUT, outer, 0)


N_AGG = 10240
RW = N_AGG // NW
SCH = 512


T_SEG = 2048
W_SEG = 4


def _seg_body(sb_ref, ids_ref, msg_ref, o_ref):
    w = pl.program_id(0)
    j = pl.program_id(1)
    base = w * RW
    idv = ids_ref[0]
    row_iota = lax.broadcasted_iota(jnp.int32, (RW, T_SEG), 0)
    sel = (idv == base + row_iota).astype(jnp.bfloat16)
    part = jnp.dot(sel, msg_ref[...].astype(jnp.bfloat16),
                   preferred_element_type=jnp.float32)

    @pl.when(j == 0)
    def _():
        o_ref[...] = part

    @pl.when(j > 0)
    def _():
        o_ref[...] = o_ref[...] + part


def _seg(msg, ids3, sb):
    grid_spec = pltpu.PrefetchScalarGridSpec(
        num_scalar_prefetch=1,
        grid=(NW, W_SEG),
        in_specs=[
            pl.BlockSpec((1, 1, T_SEG), lambda w, j, sb: (sb[w] + j, 0, 0)),
            pl.BlockSpec((T_SEG, D), lambda w, j, sb: (sb[w] + j, 0)),
        ],
        out_specs=pl.BlockSpec((RW, D), lambda w, j, sb: (w, 0)),
    )
    return pl.pallas_call(
        _seg_body,
        grid_spec=grid_spec,
        out_shape=jax.ShapeDtypeStruct((N_AGG, D), jnp.float32),
    )(sb, ids3, msg)


def _lin0_body(x_ref, w_ref, b_ref, o_ref):
    out = jax.nn.relu(
        jnp.dot(x_ref[...], w_ref[...], preferred_element_type=jnp.float32)
        + b_ref[...][0:1, :])
    o_ref[...] = jnp.tile(out, (1, 4))


def _lin0(x, wT, b8):
    return pl.pallas_call(
        _lin0_body,
        out_shape=jax.ShapeDtypeStruct((N, 128), jnp.float32),
    )(x, wT, b8)


def _eh_body(a_ref, w_ref, b_ref, o_ref):
    o_ref[...] = jax.nn.relu(
        jnp.dot(a_ref[...], w_ref[...], preferred_element_type=jnp.float32)
        + b_ref[...][0:1, :])


def _ehk(ea_p, w1T, b8):
    blk = E_PAD // 8
    return pl.pallas_call(
        _eh_body,
        grid=(8,),
        in_specs=[
            pl.BlockSpec((blk, D_EDGE), lambda i: (i, 0)),
            pl.BlockSpec((D_EDGE, EH), lambda i: (0, 0)),
            pl.BlockSpec((8, EH), lambda i: (0, 0)),
        ],
        out_specs=pl.BlockSpec((blk, EH), lambda i: (i, 0)),
        out_shape=jax.ShapeDtypeStruct((E_PAD, EH), jnp.float32),
    )(ea_p, w1T, b8)


T_MSG = 2048


def _msg_body(eh_ref, xs_ref, bm_ref, b2r_ref, o_ref):
    ehb = eh_ref[...]
    xsw = xs_ref[...]
    z = (jnp.repeat(ehb, D, axis=1)
         * jnp.tile(xsw, (1, EH // 4))).astype(jnp.bfloat16)
    o_ref[...] = (
        jnp.dot(z, bm_ref[...].astype(jnp.bfloat16),
                preferred_element_type=jnp.float32)
        + jnp.dot(xsw[:, 0:D], b2r_ref[...],
                  preferred_element_type=jnp.float32))


def _msg(eh_p, xs, bmat, b2r):
    return pl.pallas_call(
        _msg_body,
        grid=(E_PAD // T_MSG,),
        in_specs=[
            pl.BlockSpec((T_MSG, EH), lambda i: (i, 0)),
            pl.BlockSpec((T_MSG, 128), lambda i: (i, 0)),
            pl.BlockSpec((EH * D, D), lambda i: (0, 0)),
            pl.BlockSpec((D, D), lambda i: (0, 0)),
        ],
        out_specs=pl.BlockSpec((T_MSG, D), lambda i: (i, 0)),
        out_shape=jax.ShapeDtypeStruct((E_PAD, D), jnp.float32),
    )(eh_p, xs, bmat, b2r)


T_GRU = 2000


def _gru_body(agg_ref, hw_ref, wih_ref, whh_ref, bih_ref, bhh_ref,
              cb_ref, o_ref):
    m = jax.nn.relu(agg_ref[...] + cb_ref[...][0:1, :])
    gi = jnp.dot(m, wih_ref[...],
                 preferred_element_type=jnp.float32) + bih_ref[...][0:1, :]
    h = hw_ref[...][:, 0:D]
    gh = jnp.dot(h, whh_ref[...],
                 preferred_element_type=jnp.float32) + bhh_ref[...][0:1, :]
    r = jax.nn.sigmoid(gi[:, 0:D] + gh[:, 0:D])
    zg = jax.nn.sigmoid(gi[:, D:2 * D] + gh[:, D:2 * D])
    n = jnp.tanh(gi[:, 2 * D:] + r * gh[:, 2 * D:])
    o_ref[...] = jnp.tile((1.0 - zg) * n + zg * h, (1, 4))


def _gru(agg, hw, wihT, whhT, bih8, bhh8, cb8):
    return pl.pallas_call(
        _gru_body,
        grid=(N // T_GRU,),
        in_specs=[
            pl.BlockSpec((T_GRU, D), lambda i: (i, 0)),
            pl.BlockSpec((T_GRU, 128), lambda i: (i, 0)),
            pl.BlockSpec((D, 3 * D), lambda i: (0, 0)),
            pl.BlockSpec((D, 3 * D), lambda i: (0, 0)),
            pl.BlockSpec((8, 3 * D), lambda i: (0, 0)),
            pl.BlockSpec((8, 3 * D), lambda i: (0, 0)),
            pl.BlockSpec((8, D), lambda i: (0, 0)),
        ],
        out_specs=pl.BlockSpec((T_GRU, 128), lambda i: (i, 0)),
        out_shape=jax.ShapeDtypeStruct((N, 128), jnp.float32),
    )(agg, hw, wihT, whhT, bih8, bhh8, cb8)


def _s2s_body(out_ref, wih0, whh0, b0, wih1, whh1, b1, wih2, whh2, b2,
              w1_ref, rb1_ref, w2_ref, rb2_ref, o_ref):
    outv = out_ref[...][:, 0:D]

    def cell(inp, h, c, wih, whh, b):
        g = (jnp.dot(inp, wih[...], preferred_element_type=jnp.float32)
             + jnp.dot(h, whh[...], preferred_element_type=jnp.float32)
             + b[...][0:1, :])
        gi = g[:, 0:D]
        gf = g[:, D:2 * D]
        gg = g[:, 2 * D:3 * D]
        go = g[:, 3 * D:4 * D]
        c2 = jax.nn.sigmoid(gf) * c + jax.nn.sigmoid(gi) * jnp.tanh(gg)
        h2 = jax.nn.sigmoid(go) * jnp.tanh(c2)
        return h2, c2

    def step(t, carry):
        q_star, h0, c0, h1, c1, h2, c2 = carry
        h0, c0 = cell(q_star, h0, c0, wih0, whh0, b0)
        h1, c1 = cell(h0, h1, c1, wih1, whh1, b1)
        h2, c2 = cell(h1, h2, c2, wih2, whh2, b2)
        q = h2
        e = jnp.sum(outv * q, axis=1, keepdims=True)
        mx = jnp.max(e, axis=0, keepdims=True)
        al = jnp.exp(e - mx)
        ssum = jnp.sum(al, axis=0, keepdims=True)
        readout = jnp.sum(al * outv, axis=0, keepdims=True) / ssum
        q_star = jnp.concatenate([q, readout], axis=1)
        return (q_star, h0, c0, h1, c1, h2, c2)

    z1 = jnp.zeros((1, D), jnp.float32)
    init = (jnp.zeros((1, 2 * D), jnp.float32), z1, z1, z1, z1, z1, z1)
    q_star = lax.fori_loop(0, S2S_STEPS, step, init)[0]
    hmid = jax.nn.relu(
        jnp.dot(q_star, w1_ref[...], preferred_element_type=jnp.float32)
        + rb1_ref[...][0:1, :])
    res = (jnp.dot(hmid, w2_ref[...], preferred_element_type=jnp.float32)
           + rb2_ref[...][0:1, :])
    o_ref[...] = jnp.broadcast_to(res, (8, D))


def _s2s(out, args):
    return pl.pallas_call(
        _s2s_body,
        out_shape=jax.ShapeDtypeStruct((8, D), jnp.float32),
    )(out, *args)


def _pad8(b):
    return jnp.broadcast_to(b[None, :], (8, b.shape[0]))


def kernel(x, edge_index, edge_attr, lin0_W, lin0_b, enet_W1, enet_b1,
           enet_W2, enet_b2, conv_b, gru_Wih, gru_Whh, gru_bih, gru_bhh,
           lstm_Wih_0, lstm_Whh_0, lstm_bih_0, lstm_bhh_0,
           lstm_Wih_1, lstm_Whh_1, lstm_bih_1, lstm_bhh_1,
           lstm_Wih_2, lstm_Whh_2, lstm_bih_2, lstm_bhh_2,
           ro_W1, ro_b1, ro_W2, ro_b2):
    pad = E_PAD - E
    order = jnp.argsort(edge_index[1])
    src_s = edge_index[0][order]
    dst_s = edge_index[1][order]
    ea_s = edge_attr[order]
    src_p = jnp.pad(src_s, (0, pad)).reshape(E_PAD // 128, 128)
    dst_p = jnp.concatenate(
        [dst_s, jnp.full((pad,), N_AGG - 8, jnp.int32)])
    ea_p = jnp.pad(ea_s, ((0, pad), (0, 0)))
    bounds = jnp.searchsorted(
        dst_p, jnp.arange(NW, dtype=jnp.int32) * RW).astype(jnp.int32)
    sb = jnp.minimum(bounds // T_SEG, E_PAD // T_SEG - W_SEG)
    ids3 = dst_p.reshape(E_PAD // T_SEG, 1, T_SEG)

    bmat = enet_W2.reshape(D, D, EH).transpose(2, 0, 1).reshape(EH * D, D)
    b2r = enet_b2.reshape(D, D)

    out = _lin0(x, lin0_W.T, _pad8(lin0_b))
    eh = _ehk(ea_p, enet_W1.T, _pad8(enet_b1))

    gru_args = (gru_Wih.T, gru_Whh.T, _pad8(gru_bih), _pad8(gru_bhh),
                _pad8(conv_b))
    for _ in range(STEPS):
        xs = _sc_gather(out, src_p)
        msg = _msg(eh, xs, bmat, b2r)
        agg = _seg(msg, ids3, sb)
        out = _gru(agg[:N], out, *gru_args)

    s2s_args = (
        lstm_Wih_0.T, lstm_Whh_0.T, _pad8(lstm_bih_0 + lstm_bhh_0),
        lstm_Wih_1.T, lstm_Whh_1.T, _pad8(lstm_bih_1 + lstm_bhh_1),
        lstm_Wih_2.T, lstm_Whh_2.T, _pad8(lstm_bih_2 + lstm_bhh_2),
        ro_W1.T, _pad8(ro_b1), ro_W2.T, _pad8(ro_b2),
    )
    res8 = _s2s(out, s2s_args)
    return res8[0:1]


# speedup vs baseline: 2.6603x; 2.6233x over previous
import functools

import jax
import jax.numpy as jnp
from jax import lax
from jax.experimental import pallas as pl
from jax.experimental.pallas import tpu as pltpu
from jax.experimental.pallas import tpu_sc as plsc

N = 10000
E = 160000
D_IN = 128
D_EDGE = 16
D = 32
EH = 32
STEPS = 6
S2S_STEPS = 6

NC, NS = 2, 16
NW = NC * NS
E_PAD = 163840
EPW = E_PAD // NW
G_OUT = 5
IDX_R = 8
ROWS_CH = IDX_R * 128
HALF = ROWS_CH // 2
N_ACC = 10112
ZR = N_ACC // NS

_sc_mesh = plsc.VectorSubcoreMesh(core_axis_name="c", subcore_axis_name="s")


@functools.partial(
    pl.kernel,
    mesh=_sc_mesh,
    out_type=jax.ShapeDtypeStruct((E_PAD, 128), jnp.float32),
    scratch_types=[
        pltpu.VMEM((IDX_R, 128), jnp.int32),
        pltpu.VMEM((HALF, 128), jnp.float32),
        pltpu.SemaphoreType.DMA,
    ],
)
def _sc_gather(table_hbm, idx_hbm, out_hbm, idx_v, rows_v, sem):
    c = lax.axis_index("c")
    s = lax.axis_index("s")
    wid = s * NC + c
    base_row = wid * (EPW // 128)

    def outer(g, _):
        irow = base_row + g * IDX_R
        pltpu.sync_copy(idx_hbm.at[pl.ds(irow, IDX_R)], idx_v)

        def half_loop(hf, _):
            copies = [
                pltpu.async_copy(table_hbm.at[idx_v.at[hf * 4 + r4]],
                                 rows_v.at[pl.ds(r4 * 128, 128)], sem)
                for r4 in range(4)
            ]
            for cp in copies:
                cp.wait()
            pltpu.sync_copy(
                rows_v, out_hbm.at[pl.ds(irow * 128 + hf * HALF, HALF)])
            return 0

        lax.fori_loop(0, 2, half_loop, 0)
        return 0

    lax.fori_loop(0, G_OUT, outer, 0)


N_AGG = 10240
RW = N_AGG // NW
SCH = 512


T_SEG = 2048
W_SEG = 4


def _seg_body(sb_ref, ids_ref, msg_ref, o_ref):
    w = pl.program_id(0)
    j = pl.program_id(1)
    base = w * RW
    idv = ids_ref[0]
    row_iota = lax.broadcasted_iota(jnp.int32, (RW, T_SEG), 0)
    sel = (idv == base + row_iota).astype(jnp.bfloat16)
    part = jnp.dot(sel, msg_ref[...].astype(jnp.bfloat16),
                   preferred_element_type=jnp.float32)

    @pl.when(j == 0)
    def _():
        o_ref[...] = part

    @pl.when(j > 0)
    def _():
        o_ref[...] = o_ref[...] + part


def _seg(msg, ids3, sb):
    grid_spec = pltpu.PrefetchScalarGridSpec(
        num_scalar_prefetch=1,
        grid=(NW, W_SEG),
        in_specs=[
            pl.BlockSpec((1, 1, T_SEG), lambda w, j, sb: (sb[w] + j, 0, 0)),
            pl.BlockSpec((T_SEG, D), lambda w, j, sb: (sb[w] + j, 0)),
        ],
        out_specs=pl.BlockSpec((RW, D), lambda w, j, sb: (w, 0)),
    )
    return pl.pallas_call(
        _seg_body,
        grid_spec=grid_spec,
        out_shape=jax.ShapeDtypeStruct((N_AGG, D), jnp.float32),
    )(sb, ids3, msg)


def _lin0_body(x_ref, w_ref, b_ref, o_ref):
    out = jax.nn.relu(
        jnp.dot(x_ref[...], w_ref[...], preferred_element_type=jnp.float32)
        + b_ref[...][0:1, :])
    o_ref[...] = jnp.tile(out, (1, 4))


def _lin0(x, wT, b8):
    return pl.pallas_call(
        _lin0_body,
        out_shape=jax.ShapeDtypeStruct((N, 128), jnp.float32),
    )(x, wT, b8)


def _eh_body(a_ref, w_ref, b_ref, o_ref):
    o_ref[...] = jax.nn.relu(
        jnp.dot(a_ref[...], w_ref[...], preferred_element_type=jnp.float32)
        + b_ref[...][0:1, :])


def _ehk(ea_p, w1T, b8):
    blk = E_PAD // 8
    return pl.pallas_call(
        _eh_body,
        grid=(8,),
        in_specs=[
            pl.BlockSpec((blk, D_EDGE), lambda i: (i, 0)),
            pl.BlockSpec((D_EDGE, EH), lambda i: (0, 0)),
            pl.BlockSpec((8, EH), lambda i: (0, 0)),
        ],
        out_specs=pl.BlockSpec((blk, EH), lambda i: (i, 0)),
        out_shape=jax.ShapeDtypeStruct((E_PAD, EH), jnp.float32),
    )(ea_p, w1T, b8)


T_MSG = 2048


def _msg_body(eh_ref, xs_ref, bm_ref, b2r_ref, r_ref, o_ref):
    ehb = eh_ref[...]
    xsw = xs_ref[...]
    eh_rep = jnp.dot(ehb, r_ref[...], preferred_element_type=jnp.float32)
    z = (eh_rep * jnp.tile(xsw, (1, EH // 4))).astype(jnp.bfloat16)
    o_ref[...] = (
        jnp.dot(z, bm_ref[...].astype(jnp.bfloat16),
                preferred_element_type=jnp.float32)
        + jnp.dot(xsw[:, 0:D], b2r_ref[...],
                  preferred_element_type=jnp.float32))


def _msg(eh_p, xs, bmat, b2r, rmat):
    return pl.pallas_call(
        _msg_body,
        grid=(E_PAD // T_MSG,),
        in_specs=[
            pl.BlockSpec((T_MSG, EH), lambda i: (i, 0)),
            pl.BlockSpec((T_MSG, 128), lambda i: (i, 0)),
            pl.BlockSpec((EH * D, D), lambda i: (0, 0)),
            pl.BlockSpec((D, D), lambda i: (0, 0)),
            pl.BlockSpec((EH, EH * D), lambda i: (0, 0)),
        ],
        out_specs=pl.BlockSpec((T_MSG, D), lambda i: (i, 0)),
        out_shape=jax.ShapeDtypeStruct((E_PAD, D), jnp.float32),
    )(eh_p, xs, bmat, b2r, rmat)


T_GRU = 2000


def _gru_body(agg_ref, hw_ref, wih_ref, whh_ref, bih_ref, bhh_ref,
              cb_ref, o_ref):
    m = jax.nn.relu(agg_ref[...] + cb_ref[...][0:1, :])
    gi = jnp.dot(m, wih_ref[...],
                 preferred_element_type=jnp.float32) + bih_ref[...][0:1, :]
    h = hw_ref[...][:, 0:D]
    gh = jnp.dot(h, whh_ref[...],
                 preferred_element_type=jnp.float32) + bhh_ref[...][0:1, :]
    r = jax.nn.sigmoid(gi[:, 0:D] + gh[:, 0:D])
    zg = jax.nn.sigmoid(gi[:, D:2 * D] + gh[:, D:2 * D])
    n = jnp.tanh(gi[:, 2 * D:] + r * gh[:, 2 * D:])
    o_ref[...] = jnp.tile((1.0 - zg) * n + zg * h, (1, 4))


def _gru(agg, hw, wihT, whhT, bih8, bhh8, cb8):
    return pl.pallas_call(
        _gru_body,
        grid=(N // T_GRU,),
        in_specs=[
            pl.BlockSpec((T_GRU, D), lambda i: (i, 0)),
            pl.BlockSpec((T_GRU, 128), lambda i: (i, 0)),
            pl.BlockSpec((D, 3 * D), lambda i: (0, 0)),
            pl.BlockSpec((D, 3 * D), lambda i: (0, 0)),
            pl.BlockSpec((8, 3 * D), lambda i: (0, 0)),
            pl.BlockSpec((8, 3 * D), lambda i: (0, 0)),
            pl.BlockSpec((8, D), lambda i: (0, 0)),
        ],
        out_specs=pl.BlockSpec((T_GRU, 128), lambda i: (i, 0)),
        out_shape=jax.ShapeDtypeStruct((N, 128), jnp.float32),
    )(agg, hw, wihT, whhT, bih8, bhh8, cb8)


def _s2s_body(out_ref, wih0, whh0, b0, wih1, whh1, b1, wih2, whh2, b2,
              w1_ref, rb1_ref, w2_ref, rb2_ref, o_ref):
    outv = out_ref[...][:, 0:D]

    def cell(inp, h, c, wih, whh, b):
        g = (jnp.dot(inp, wih[...], preferred_element_type=jnp.float32)
             + jnp.dot(h, whh[...], preferred_element_type=jnp.float32)
             + b[...][0:1, :])
        gi = g[:, 0:D]
        gf = g[:, D:2 * D]
        gg = g[:, 2 * D:3 * D]
        go = g[:, 3 * D:4 * D]
        c2 = jax.nn.sigmoid(gf) * c + jax.nn.sigmoid(gi) * jnp.tanh(gg)
        h2 = jax.nn.sigmoid(go) * jnp.tanh(c2)
        return h2, c2

    def step(t, carry):
        q_star, h0, c0, h1, c1, h2, c2 = carry
        h0, c0 = cell(q_star, h0, c0, wih0, whh0, b0)
        h1, c1 = cell(h0, h1, c1, wih1, whh1, b1)
        h2, c2 = cell(h1, h2, c2, wih2, whh2, b2)
        q = h2
        e = jnp.sum(outv * q, axis=1, keepdims=True)
        mx = jnp.max(e, axis=0, keepdims=True)
        al = jnp.exp(e - mx)
        ssum = jnp.sum(al, axis=0, keepdims=True)
        readout = jnp.sum(al * outv, axis=0, keepdims=True) / ssum
        q_star = jnp.concatenate([q, readout], axis=1)
        return (q_star, h0, c0, h1, c1, h2, c2)

    z1 = jnp.zeros((1, D), jnp.float32)
    init = (jnp.zeros((1, 2 * D), jnp.float32), z1, z1, z1, z1, z1, z1)
    q_star = lax.fori_loop(0, S2S_STEPS, step, init)[0]
    hmid = jax.nn.relu(
        jnp.dot(q_star, w1_ref[...], preferred_element_type=jnp.float32)
        + rb1_ref[...][0:1, :])
    res = (jnp.dot(hmid, w2_ref[...], preferred_element_type=jnp.float32)
           + rb2_ref[...][0:1, :])
    o_ref[...] = jnp.broadcast_to(res, (8, D))


def _s2s(out, args):
    return pl.pallas_call(
        _s2s_body,
        out_shape=jax.ShapeDtypeStruct((8, D), jnp.float32),
    )(out, *args)


def _pad8(b):
    return jnp.broadcast_to(b[None, :], (8, b.shape[0]))


def kernel(x, edge_index, edge_attr, lin0_W, lin0_b, enet_W1, enet_b1,
           enet_W2, enet_b2, conv_b, gru_Wih, gru_Whh, gru_bih, gru_bhh,
           lstm_Wih_0, lstm_Whh_0, lstm_bih_0, lstm_bhh_0,
           lstm_Wih_1, lstm_Whh_1, lstm_bih_1, lstm_bhh_1,
           lstm_Wih_2, lstm_Whh_2, lstm_bih_2, lstm_bhh_2,
           ro_W1, ro_b1, ro_W2, ro_b2):
    pad = E_PAD - E
    order = jnp.argsort(edge_index[1])
    src_s = edge_index[0][order]
    dst_s = edge_index[1][order]
    ea_s = edge_attr[order]
    src_p = jnp.pad(src_s, (0, pad)).reshape(E_PAD // 128, 128)
    dst_p = jnp.concatenate(
        [dst_s, jnp.full((pad,), N_AGG - 8, jnp.int32)])
    ea_p = jnp.pad(ea_s, ((0, pad), (0, 0)))
    bounds = jnp.searchsorted(
        dst_p, jnp.arange(NW, dtype=jnp.int32) * RW).astype(jnp.int32)
    sb = jnp.minimum(bounds // T_SEG, E_PAD // T_SEG - W_SEG)
    ids3 = dst_p.reshape(E_PAD // T_SEG, 1, T_SEG)

    bmat = enet_W2.reshape(D, D, EH).transpose(2, 0, 1).reshape(EH * D, D)
    b2r = enet_b2.reshape(D, D)
    rmat = jnp.kron(jnp.eye(EH, dtype=jnp.float32),
                    jnp.ones((1, D), jnp.float32))

    out = _lin0(x, lin0_W.T, _pad8(lin0_b))
    eh = _ehk(ea_p, enet_W1.T, _pad8(enet_b1))

    gru_args = (gru_Wih.T, gru_Whh.T, _pad8(gru_bih), _pad8(gru_bhh),
                _pad8(conv_b))
    for _ in range(STEPS):
        xs = _sc_gather(out, src_p)
        msg = _msg(eh, xs, bmat, b2r, rmat)
        agg = _seg(msg, ids3, sb)
        out = _gru(agg[:N], out, *gru_args)

    s2s_args = (
        lstm_Wih_0.T, lstm_Whh_0.T, _pad8(lstm_bih_0 + lstm_bhh_0),
        lstm_Wih_1.T, lstm_Whh_1.T, _pad8(lstm_bih_1 + lstm_bhh_1),
        lstm_Wih_2.T, lstm_Whh_2.T, _pad8(lstm_bih_2 + lstm_bhh_2),
        ro_W1.T, _pad8(ro_b1), ro_W2.T, _pad8(ro_b2),
    )
    res8 = _s2s(out, s2s_args)
    return res8[0:1]
